# Initial kernel scaffold; baseline (speedup 1.0000x reference)
import jax
import jax.numpy as jnp
from jax.experimental import pallas as pl


def kernel(x, batch, W, b):
    raise NotImplementedError("write your pallas kernel here")


# speedup vs baseline: 2.4469x; 2.4469x over previous
import functools

import jax
import jax.numpy as jnp
from jax import lax
from jax.experimental import pallas as pl
from jax.experimental.pallas import tpu as pltpu
from jax.experimental.pallas import tpu_sc as plsc

N = 100000
D = 128
C = 10
S = 2048

NC = 2
NS = 16
L = 16
NF = 8
NR = 4

BLK = 160
GPB = BLK // L
NBLK = N // BLK
ITERS = (NBLK + NR - 1) // NR


def _sc_body(x_hbm, batch_hbm, partial_hbm, counts_hbm,
             acc, cnt, xb0, xb1, bb0, bb1, sx0, sx1, sb0, sb1):
    c = lax.axis_index("c")
    s = lax.axis_index("s")
    wid = s * NC + c
    f = wid % NF
    r = wid // NF
    col0 = f * L

    iota = lax.iota(jnp.int32, L)
    zeros = jnp.zeros((L,), jnp.float32)
    ones = jnp.ones((L,), jnp.float32)
    lane0 = iota == 0

    @pl.loop(0, S)
    def _(i):
        acc[i, :] = zeros

    @pl.loop(0, S // L)
    def _(i):
        cnt[pl.ds(i * L, L)] = zeros

    xbufs = (xb0, xb1)
    bbufs = (bb0, bb1)
    xsems = (sx0, sx1)
    bsems = (sb0, sb1)

    def x_copy(it, d):
        b = it * NR + r
        row0 = b * BLK
        return pltpu.make_async_copy(
            x_hbm.at[pl.ds(row0, BLK), pl.ds(col0, L)], xbufs[d], xsems[d])

    def b_copy(it, d):
        b = it * NR + r
        row0 = b * BLK
        return pltpu.make_async_copy(
            batch_hbm.at[pl.ds(row0, BLK)], bbufs[d], bsems[d])

    for d in range(2):
        x_copy(d, d).start()
        b_copy(d, d).start()

    def process(it, d):
        b = it * NR + r
        x_copy(it, d).wait()
        b_copy(it, d).wait()
        xb = xbufs[d]
        bb = bbufs[d]
        for g in range(GPB):
            bv = bb[pl.ds(g * L, L)]
            segs = []
            for i in range(L):
                seg = bv.at[jnp.full((L,), i, jnp.int32)].get(
                    mode="promise_in_bounds")
                segs.append(seg)
                plsc.addupdate_scatter(acc, [seg, iota], xb[g * L + i, :])
            gid = b * GPB + g

            @pl.when(gid % NF == f)
            def _():
                for i in range(L):
                    plsc.addupdate_scatter(cnt, [segs[i]], ones, mask=lane0)

    @pl.loop(0, ITERS + 1, step=2)
    def _(k):
        for d in range(2):
            it = k + d
            valid = (it * NR + r) < NBLK

            @pl.when(valid)
            def _():
                process(it, d)

            @pl.when(((it + 2) * NR + r) < NBLK)
            def _():
                x_copy(it + 2, d).start()
                b_copy(it + 2, d).start()

    pltpu.sync_copy(acc, partial_hbm.at[r, :, pl.ds(col0, L)])
    pltpu.sync_copy(cnt, counts_hbm.at[wid])


@jax.jit
def _sc_segment_sum(x, batch):
    mesh = plsc.VectorSubcoreMesh(core_axis_name="c", subcore_axis_name="s")
    return pl.kernel(
        _sc_body,
        out_type=[
            jax.ShapeDtypeStruct((NR, S, D), jnp.float32),
            jax.ShapeDtypeStruct((NC * NS, S), jnp.float32),
        ],
        mesh=mesh,
        compiler_params=pltpu.CompilerParams(use_tc_tiling_on_sc=False,
                                             needs_layout_passes=False),
        scratch_types=[
            pltpu.VMEM((S, L), jnp.float32),
            pltpu.VMEM((S,), jnp.float32),
            pltpu.VMEM((BLK, L), jnp.float32),
            pltpu.VMEM((BLK, L), jnp.float32),
            pltpu.VMEM((BLK,), jnp.int32),
            pltpu.VMEM((BLK,), jnp.int32),
            pltpu.SemaphoreType.DMA,
            pltpu.SemaphoreType.DMA,
            pltpu.SemaphoreType.DMA,
            pltpu.SemaphoreType.DMA,
        ],
    )(x, batch)


TCB = 256


def _tc_body(p_ref, cnt_ref, w_ref, b_ref, o_ref):
    p = p_ref[0] + p_ref[1] + p_ref[2] + p_ref[3]
    counts = jnp.sum(cnt_ref[...], axis=0)
    o_ref[...] = (jnp.dot(p, w_ref[...], preferred_element_type=jnp.float32)
                  + counts[:, None] * b_ref[...])


@jax.jit
def _tc_head(partial, counts, W, b2):
    return pl.pallas_call(
        _tc_body,
        grid=(S // TCB,),
        in_specs=[
            pl.BlockSpec((NR, TCB, D), lambda i: (0, i, 0)),
            pl.BlockSpec((NC * NS, TCB), lambda i: (0, i)),
            pl.BlockSpec((D, C), lambda i: (0, 0)),
            pl.BlockSpec((1, C), lambda i: (0, 0)),
        ],
        out_specs=pl.BlockSpec((TCB, C), lambda i: (i, 0)),
        out_shape=jax.ShapeDtypeStruct((S, C), jnp.float32),
    )(partial, counts, W, b2)


def kernel(x, batch, W, b):
    batch_i = batch.astype(jnp.int32)
    partial, counts = _sc_segment_sum(x, batch_i)
    return _tc_head(partial, counts, W, b.reshape(1, C))


# speedup vs baseline: 4.0307x; 1.6473x over previous
import jax
import jax.numpy as jnp
from jax import lax
from jax.experimental import pallas as pl
from jax.experimental.pallas import tpu as pltpu
from jax.experimental.pallas import tpu_sc as plsc

N = 100000
D = 128
C = 10
S = 2048
S_PAD = 2064

NC = 2
NS = 16
L = 16
NW = NC * NS

BLK = 125
BLK_PAD = 128
NBLK = N // (NW * BLK)
ROWS_W = BLK * NBLK
ZR = S // NS


def _sc_body(x_hbm, bidx_hbm, partial_hbm, cnt_hbm,
             acc, xb0, xb1, ib0, ib1, cnt,
             sf0, sf1, si0, si1, ss0, ss1):
    cid = lax.axis_index("c")
    sid = lax.axis_index("s")
    wid = cid * NS + sid

    iota = lax.iota(jnp.int32, L)
    zeros = jnp.zeros((L,), jnp.float32)
    ones = jnp.ones((L,), jnp.float32)
    lane0 = iota == 0

    xbufs = (xb0, xb1)
    ibufs = (ib0, ib1)
    fsems = (sf0, sf1)
    isems = (si0, si1)
    ssems = (ss0, ss1)

    def fill(it, d):
        row0 = wid * ROWS_W + it * BLK
        pltpu.async_copy(
            x_hbm.at[pl.ds(row0, BLK), :], xbufs[d].at[pl.ds(0, BLK), :],
            fsems[d])
        pltpu.async_copy(bidx_hbm.at[wid * NBLK + it, :], ibufs[d], isems[d])

    def wait_fill(d):
        pltpu.make_async_copy(
            x_hbm.at[pl.ds(0, BLK), :], xbufs[d].at[pl.ds(0, BLK), :],
            fsems[d]).wait()
        pltpu.make_async_copy(bidx_hbm.at[0, :], ibufs[d], isems[d]).wait()

    def scatter_start(d):
        pltpu.async_copy(xbufs[d], acc.at[ibufs[d]], ssems[d], add=True)

    def wait_scatter(d):
        pltpu.make_async_copy(xbufs[d], acc.at[ibufs[d]], ssems[d]).wait()

    def count_block(d):
        ib = ibufs[d]

        @pl.loop(0, BLK_PAD // L)
        def _(g):
            bv = ib[pl.ds(g * L, L)]
            for i in range(L):
                seg = bv.at[jnp.full((L,), i, jnp.int32)].get(
                    mode="promise_in_bounds")
                plsc.addupdate_scatter(cnt, [seg], ones, mask=lane0)

    @pl.loop(0, ZR)
    def _(i):
        for j in range(D // L):
            xb0[i, pl.ds(j * L, L)] = zeros

    @pl.loop(0, S_PAD // L)
    def _(i):
        cnt[pl.ds(i * L, L)] = zeros

    pltpu.sync_copy(xb0, acc.at[pl.ds(sid * ZR, ZR), :])

    @pl.when(sid == 0)
    def _():
        pltpu.sync_copy(xb0.at[pl.ds(0, S_PAD - S), :],
                        acc.at[pl.ds(S, S_PAD - S), :])

    plsc.subcore_barrier()

    @pl.loop(0, NBLK)
    def _(it):
        row0 = wid * ROWS_W + it * BLK
        pltpu.sync_copy(x_hbm.at[pl.ds(row0, BLK), :],
                        xb0.at[pl.ds(0, BLK), :])
        pltpu.sync_copy(bidx_hbm.at[wid * NBLK + it, :], ib0)
        pltpu.sync_copy(xb0, acc.at[ib0], add=True)
        count_block(0)

    plsc.subcore_barrier()

    pltpu.sync_copy(acc.at[pl.ds(sid * ZR, ZR), :],
                    partial_hbm.at[cid, pl.ds(sid * ZR, ZR), :])
    pltpu.sync_copy(cnt.at[pl.ds(0, S)], cnt_hbm.at[wid])


@jax.jit
def _sc_segment_sum(x, bidx):
    mesh = plsc.VectorSubcoreMesh(core_axis_name="c", subcore_axis_name="s")
    return pl.kernel(
        _sc_body,
        out_type=[
            jax.ShapeDtypeStruct((NC, S, D), jnp.float32),
            jax.ShapeDtypeStruct((NW, S), jnp.float32),
        ],
        mesh=mesh,
        compiler_params=pltpu.CompilerParams(use_tc_tiling_on_sc=False,
                                             needs_layout_passes=False),
        scratch_types=[
            pltpu.VMEM_SHARED((S_PAD, D), jnp.float32),
            pltpu.VMEM((BLK_PAD, D), jnp.float32),
            pltpu.VMEM((BLK_PAD, D), jnp.float32),
            pltpu.VMEM((BLK_PAD,), jnp.int32),
            pltpu.VMEM((BLK_PAD,), jnp.int32),
            pltpu.VMEM((S_PAD,), jnp.float32),
            pltpu.SemaphoreType.DMA,
            pltpu.SemaphoreType.DMA,
            pltpu.SemaphoreType.DMA,
            pltpu.SemaphoreType.DMA,
            pltpu.SemaphoreType.DMA,
            pltpu.SemaphoreType.DMA,
        ],
    )(x, bidx)


TCB = 256


def _tc_body(p_ref, cnt_ref, w_ref, b_ref, o_ref):
    p = p_ref[0] + p_ref[1]
    counts = jnp.sum(cnt_ref[...], axis=0)
    o_ref[...] = (jnp.dot(p, w_ref[...], preferred_element_type=jnp.float32)
                  + counts[:, None] * b_ref[...])


@jax.jit
def _tc_head(partial, counts, W, b2):
    return pl.pallas_call(
        _tc_body,
        grid=(S // TCB,),
        in_specs=[
            pl.BlockSpec((NC, TCB, D), lambda i: (0, i, 0)),
            pl.BlockSpec((NW, TCB), lambda i: (0, i)),
            pl.BlockSpec((D, C), lambda i: (0, 0)),
            pl.BlockSpec((1, C), lambda i: (0, 0)),
        ],
        out_specs=pl.BlockSpec((TCB, C), lambda i: (i, 0)),
        out_shape=jax.ShapeDtypeStruct((S, C), jnp.float32),
    )(partial, counts, W, b2)


def kernel(x, batch, W, b):
    batch_i = batch.astype(jnp.int32)
    bidx = jnp.full((NW * NBLK, BLK_PAD), S, jnp.int32)
    bidx = bidx.at[:, :BLK].set(batch_i.reshape(NW * NBLK, BLK))
    partial, counts = _sc_segment_sum(x, bidx)
    return _tc_head(partial, counts, W, b.reshape(1, C))
